# baseline (speedup 1.0000x reference)
import functools
import math

import jax
import jax.numpy as jnp
from jax.experimental import pallas as pl
from jax.experimental.pallas import tpu as pltpu
from jax.experimental.pallas import tpu_sc as plsc

FACIL_RATE = 2.0
FIRE_THRESH = 0.85
MAX_GATE = 8.0

_INV_SQRT2 = 1.0 / math.sqrt(2.0)

_ROW_BLOCK = 4096
_SCALE_BLOCK = 2048
_D = 1024


def _gelu(x):
    h = 0.5 * x
    return h + h * jax.lax.erf(x * _INV_SQRT2)


def _sum_kernel(x_ref, out_ref, acc_ref):
    i = pl.program_id(0)

    def body(j, carry):
        s1, s2 = carry
        xx = x_ref[pl.ds(j * 8, 8), :]
        e = jax.lax.erf(xx * _INV_SQRT2)
        return (s1 + xx, s2 + xx * e)

    zero = jnp.zeros((8, _D), jnp.float32)
    s1, s2 = jax.lax.fori_loop(
        0, _ROW_BLOCK // 8, body, (zero, zero), unroll=4)
    part = 0.5 * (s1 + s2)

    @pl.when(i == 0)
    def _init():
        acc_ref[...] = part

    @pl.when(i > 0)
    def _acc():
        acc_ref[...] += part

    @pl.when(i == pl.num_programs(0) - 1)
    def _emit():
        out_ref[...] = acc_ref[...]


def _column_sums(x2):
    n_rows = x2.shape[0]
    grid = n_rows // _ROW_BLOCK
    return pl.pallas_call(
        _sum_kernel,
        grid=(grid,),
        in_specs=[pl.BlockSpec((_ROW_BLOCK, _D), lambda i: (i, 0))],
        out_specs=pl.BlockSpec((8, _D), lambda i: (0, 0)),
        out_shape=jax.ShapeDtypeStruct((8, _D), jnp.float32),
        scratch_shapes=[pltpu.VMEM((8, _D), jnp.float32)],
        compiler_params=pltpu.CompilerParams(
            dimension_semantics=("arbitrary",)),
    )(x2)


def _gate_kernel(sums_ref, buf_ref, facil_ref, mask_ref, scal_ref,
                 out_ref, *, n_rows):
    log_k_gate = scal_ref[0, 0]
    log_sharpness = scal_ref[0, 1]
    k_gate = jnp.clip(jnp.exp(log_k_gate), 0.01, 5.0)
    sharpness = jnp.clip(jnp.exp(log_sharpness), 0.5, 20.0)

    m = jnp.sum(sums_ref[...], axis=0, keepdims=True) / jnp.float32(n_rows)
    norm = jnp.sqrt(jnp.sum(m * m))
    q = m / jnp.maximum(norm, 1e-12)
    sims = jax.lax.dot_general(
        buf_ref[...], q,
        dimension_numbers=(((1,), (1,)), ((), ())),
        preferred_element_type=jnp.float32)
    mask = mask_ref[...] > 0.5
    sims_masked = jnp.where(mask, sims, -1.0)
    sim_nearest = jnp.max(sims_masked)
    iota = jax.lax.broadcasted_iota(jnp.int32, sims.shape, 0)
    nearest_idx = jnp.min(jnp.where(sims_masked == sim_nearest, iota, 2**30))
    n_valid = jnp.sum(mask.astype(jnp.float32))
    sim_at_nearest = jnp.sum(jnp.where(iota == nearest_idx, sims, 0.0))
    sum_others = jnp.sum(jnp.where(mask, sims, 0.0)) - sim_at_nearest
    mean_others = sum_others / jnp.maximum(n_valid - 1.0, 1.0)
    contrast = jnp.where(n_valid > 1.0, sim_nearest - mean_others, 0.0)
    fire_mult = jnp.where(sim_nearest > FIRE_THRESH, FACIL_RATE, 1.0)
    facil_level = jnp.sum(
        jnp.where(iota == nearest_idx, facil_ref[...], 0.0)) * fire_mult
    selectivity = jax.nn.sigmoid(sharpness * contrast)
    gate = jnp.minimum(1.0 + k_gate * (facil_level - 1.0) * selectivity,
                       MAX_GATE)
    out_ref[0, 0] = gate


def _compute_gate(sums, buf, facil, mask_f, scal, n_rows):
    return pl.pallas_call(
        functools.partial(_gate_kernel, n_rows=n_rows),
        in_specs=[
            pl.BlockSpec(memory_space=pltpu.VMEM),
            pl.BlockSpec(memory_space=pltpu.VMEM),
            pl.BlockSpec(memory_space=pltpu.VMEM),
            pl.BlockSpec(memory_space=pltpu.VMEM),
            pl.BlockSpec(memory_space=pltpu.SMEM),
        ],
        out_specs=pl.BlockSpec(memory_space=pltpu.SMEM),
        out_shape=jax.ShapeDtypeStruct((1, 1), jnp.float32),
    )(sums, buf, facil, mask_f, scal)


_N_BUF = 512
_N_SC_WORKERS = 16
_ROWS_PER_W = _N_BUF // _N_SC_WORKERS
_L = 16


def _full(v, dtype=jnp.float32):
    return jnp.full((_L,), v, dtype=dtype)


def _vsum(vec):
    s = vec[0]
    for i in range(1, _L):
        s = s + vec[i]
    return s


def _vmax(vec):
    s = vec[0]
    for i in range(1, _L):
        s = jnp.maximum(s, vec[i])
    return s


def _sc_gate_body(n_rows, sums_hbm, buf_hbm, facil_hbm, mask_hbm, scal_hbm,
                  out_hbm, sums_v, buf_v, maskf_v, scal_v, facil_v, m_v,
                  stage_v, all_v, out_v, shared):
    w = jax.lax.axis_index("s")
    base_row = w * _ROWS_PER_W

    pltpu.sync_copy(sums_hbm, sums_v)
    pltpu.sync_copy(buf_hbm.at[pl.ds(base_row * 1, _ROWS_PER_W), :], buf_v)
    pltpu.sync_copy(mask_hbm.at[pl.ds(base_row * 1, _ROWS_PER_W)], maskf_v)
    pltpu.sync_copy(scal_hbm, scal_v)

    @pl.when(w == 0)
    def _copy_facil():
        pltpu.sync_copy(facil_hbm, facil_v)

    scale = 1.0 / float(n_rows)

    def m_body(c, nrm):
        b = c * _L
        s = sums_v[0, pl.ds(b, _L)]
        for r in range(1, 8):
            s = s + sums_v[r, pl.ds(b, _L)]
        mv = s * scale
        m_v[pl.ds(b, _L)] = mv
        return nrm + mv * mv

    nrm = jax.lax.fori_loop(0, _D // _L, m_body,
                            jnp.zeros((_L,), jnp.float32), unroll=4)
    nsq = _vsum(nrm)
    nsqc = jnp.maximum(nsq, jnp.float32(1e-30))
    i = jax.lax.bitcast_convert_type(nsqc, jnp.int32)
    i = jnp.int32(0x5F3759DF) - jax.lax.shift_right_logical(i, 1)
    y = jax.lax.bitcast_convert_type(i, jnp.float32)
    for _ in range(4):
        y = y * (1.5 - 0.5 * nsqc * y * y)
    norm = nsqc * y
    invv = 1.0 / jnp.maximum(_full(0.0) + norm, _full(1e-12))

    lanes = jax.lax.broadcasted_iota(jnp.int32, (_L,), 0)
    lmax = jnp.float32(-2.0)
    lidx = jnp.int32(0)
    lsum = jnp.float32(0.0)
    lcnt = jnp.float32(0.0)
    for k in range(_ROWS_PER_W // _L):
        simvec = jnp.zeros((_L,), jnp.float32)
        for j in range(_L):
            r = k * _L + j

            def dot_body(c, acc, _r=r):
                b = c * _L
                return acc + buf_v[_r, pl.ds(b, _L)] * m_v[pl.ds(b, _L)]

            acc = jax.lax.fori_loop(0, _D // _L, dot_body,
                                    jnp.zeros((_L,), jnp.float32), unroll=8)
            simvec = jnp.where(lanes == j, _full(0.0) + _vsum(acc),
                               simvec)

        sv = simvec * invv
        mk = maskf_v[pl.ds(k * _L, _L)] > 0.5
        sm = jnp.where(mk, sv, _full(-1.0))
        cmax = _vmax(sm)
        lanei = jnp.int32(_L - 1)
        for i2 in range(_L - 1, -1, -1):
            lanei = jnp.where(sm[i2] == cmax, jnp.int32(i2), lanei)
        cidx = base_row + k * _L + lanei
        csum = _vsum(jnp.where(mk, sv, _full(0.0)))
        ccnt = _vsum(maskf_v[pl.ds(k * _L, _L)])
        take = cmax > lmax
        lmax = jnp.where(take, cmax, lmax)
        lidx = jnp.where(take, cidx, lidx)
        lsum = lsum + csum
        lcnt = lcnt + ccnt

    statv = jnp.where(lanes == 0, _full(0.0) + lmax,
            jnp.where(lanes == 1, _full(0.0) + lidx.astype(jnp.float32),
            jnp.where(lanes == 2, _full(0.0) + lsum,
            jnp.where(lanes == 3, _full(0.0) + lcnt, _full(0.0)))))
    stage_v[...] = statv
    pltpu.sync_copy(stage_v, shared.at[w])
    plsc.subcore_barrier()

    @pl.when(w == 0)
    def _finalize():
        pltpu.sync_copy(shared, all_v)
        gmax = jnp.float32(-2.0)
        gidx = jnp.int32(0)
        gsum = jnp.float32(0.0)
        gcnt = jnp.float32(0.0)
        for ww in range(_N_SC_WORKERS):
            row = all_v[ww, :]
            wmax = row[0]
            widx = row[1]
            gsum = gsum + row[2]
            gcnt = gcnt + row[3]
            take = wmax > gmax
            gmax = jnp.where(take, wmax, gmax)
            gidx = jnp.where(take, widx.astype(jnp.int32), gidx)

        fbase = jnp.bitwise_and(gidx, jnp.int32(~(_L - 1)))
        fwin = facil_v[pl.ds(fbase, _L)]
        flane = gidx - fbase
        fval = fwin[0]
        for i3 in range(1, _L):
            fval = jnp.where(flane == i3, fwin[i3], fval)
        fvec = _full(0.0) + fval
        gmaxv = _full(0.0) + gmax
        gsumv = _full(0.0) + gsum
        gcntv = _full(0.0) + gcnt
        meano = (gsumv - gmaxv) / jnp.maximum(gcntv - 1.0, _full(1.0))
        contrast = jnp.where(gcntv > 1.0, gmaxv - meano, _full(0.0))
        scal_vec = scal_v[...]
        kg = jnp.clip(jnp.exp(_full(0.0) + scal_vec[0]), 0.01, 5.0)
        sh = jnp.clip(jnp.exp(_full(0.0) + scal_vec[1]), 0.5, 20.0)
        fire = jnp.where(gmaxv > FIRE_THRESH, _full(FACIL_RATE), _full(1.0))
        flevel = fvec * fire
        sel = 1.0 / (1.0 + jnp.exp(-(sh * contrast)))
        gate = jnp.minimum(1.0 + kg * (flevel - 1.0) * sel, _full(MAX_GATE))
        out_v[...] = gate
        pltpu.sync_copy(out_v, out_hbm)


def _compute_gate_sc(sums, buf, facil, mask_f, scal, n_rows):
    mesh = plsc.VectorSubcoreMesh(
        core_axis_name="c", subcore_axis_name="s", num_cores=1)
    body = functools.partial(_sc_gate_body, n_rows)
    fn = pl.kernel(
        body,
        out_type=jax.ShapeDtypeStruct((_L,), jnp.float32),
        mesh=mesh,
        scratch_types=[
            pltpu.VMEM((8, _D), jnp.float32),
            pltpu.VMEM((_ROWS_PER_W, _D), jnp.float32),
            pltpu.VMEM((_ROWS_PER_W,), jnp.float32),
            pltpu.VMEM((_L,), jnp.float32),
            pltpu.VMEM((_N_BUF,), jnp.float32),
            pltpu.VMEM((_D,), jnp.float32),
            pltpu.VMEM((_L,), jnp.float32),
            pltpu.VMEM((_N_SC_WORKERS, _L), jnp.float32),
            pltpu.VMEM((_L,), jnp.float32),
            pltpu.VMEM_SHARED((_N_SC_WORKERS, _L), jnp.float32),
        ],
    )
    return fn(sums, buf, facil, mask_f, scal)


def _scale_kernel(gate_ref, x_ref, out_ref):
    x = x_ref[...]
    a = (0.5 * gate_ref[0, 0]) * x
    out_ref[...] = a + a * jax.lax.erf(x * _INV_SQRT2)


def _scale(x2, gate):
    n_rows = x2.shape[0]
    grid = n_rows // _SCALE_BLOCK
    return pl.pallas_call(
        _scale_kernel,
        grid=(grid,),
        in_specs=[
            pl.BlockSpec(memory_space=pltpu.SMEM),
            pl.BlockSpec((_SCALE_BLOCK, _D), lambda i: (i, 0)),
        ],
        out_specs=pl.BlockSpec((_SCALE_BLOCK, _D), lambda i: (i, 0)),
        out_shape=jax.ShapeDtypeStruct((n_rows, _D), jnp.float32),
        compiler_params=pltpu.CompilerParams(
            dimension_semantics=("parallel",)),
    )(gate, x2)


def kernel(x, log_k_gate, log_sharpness, buf, facil, mask):
    orig_shape = x.shape
    x2 = x.reshape(-1, x.shape[-1])
    n_rows = x2.shape[0]

    sums = _column_sums(x2)

    scal = jnp.zeros((_L,), jnp.float32)
    scal = scal.at[0].set(log_k_gate).at[1].set(log_sharpness)
    mask_f = mask.astype(jnp.float32)
    gate16 = _compute_gate_sc(sums, buf, facil, mask_f, scal, n_rows)
    gate = gate16[0:1].reshape(1, 1)

    out = _scale(x2, gate)
    return out.reshape(orig_shape)


# speedup vs baseline: 1.0231x; 1.0231x over previous
import functools
import math

import jax
import jax.numpy as jnp
from jax.experimental import pallas as pl
from jax.experimental.pallas import tpu as pltpu
from jax.experimental.pallas import tpu_sc as plsc

FACIL_RATE = 2.0
FIRE_THRESH = 0.85
MAX_GATE = 8.0

_INV_SQRT2 = 1.0 / math.sqrt(2.0)

_ROW_BLOCK = 4096
_SCALE_BLOCK = 2048
_D = 1024
_N_ROWS = 4 * 8192


def _gelu(x):
    h = 0.5 * x
    return h + h * jax.lax.erf(x * _INV_SQRT2)


def _sum_kernel(x_ref, out_ref, acc_ref):
    i = pl.program_id(0)

    def body(j, carry):
        s1, s2 = carry
        xx = x_ref[pl.ds(j * 8, 8), :]
        e = jax.lax.erf(xx * _INV_SQRT2)
        return (s1 + xx, s2 + xx * e)

    zero = jnp.zeros((8, _D), jnp.float32)
    s1, s2 = jax.lax.fori_loop(
        0, _ROW_BLOCK // 8, body, (zero, zero), unroll=4)
    part = 0.5 * (s1 + s2)

    @pl.when(i == 0)
    def _init():
        acc_ref[...] = part

    @pl.when(i > 0)
    def _acc():
        acc_ref[...] += part

    @pl.when(i == pl.num_programs(0) - 1)
    def _emit():
        total = acc_ref[...]
        m = jnp.sum(total, axis=0, keepdims=True) * (1.0 / float(_N_ROWS))
        norm = jnp.sqrt(jnp.sum(m * m))
        out_ref[...] = m / jnp.maximum(norm, 1e-12)


def _column_sums(x2):
    n_rows = x2.shape[0]
    grid = n_rows // _ROW_BLOCK
    return pl.pallas_call(
        _sum_kernel,
        grid=(grid,),
        in_specs=[pl.BlockSpec((_ROW_BLOCK, _D), lambda i: (i, 0))],
        out_specs=pl.BlockSpec((1, _D), lambda i: (0, 0)),
        out_shape=jax.ShapeDtypeStruct((1, _D), jnp.float32),
        scratch_shapes=[pltpu.VMEM((8, _D), jnp.float32)],
        compiler_params=pltpu.CompilerParams(
            dimension_semantics=("arbitrary",)),
    )(x2)


def _gate_kernel(sums_ref, buf_ref, facil_ref, mask_ref, scal_ref,
                 out_ref, *, n_rows):
    log_k_gate = scal_ref[0, 0]
    log_sharpness = scal_ref[0, 1]
    k_gate = jnp.clip(jnp.exp(log_k_gate), 0.01, 5.0)
    sharpness = jnp.clip(jnp.exp(log_sharpness), 0.5, 20.0)

    m = jnp.sum(sums_ref[...], axis=0, keepdims=True) / jnp.float32(n_rows)
    norm = jnp.sqrt(jnp.sum(m * m))
    q = m / jnp.maximum(norm, 1e-12)
    sims = jax.lax.dot_general(
        buf_ref[...], q,
        dimension_numbers=(((1,), (1,)), ((), ())),
        preferred_element_type=jnp.float32)
    mask = mask_ref[...] > 0.5
    sims_masked = jnp.where(mask, sims, -1.0)
    sim_nearest = jnp.max(sims_masked)
    iota = jax.lax.broadcasted_iota(jnp.int32, sims.shape, 0)
    nearest_idx = jnp.min(jnp.where(sims_masked == sim_nearest, iota, 2**30))
    n_valid = jnp.sum(mask.astype(jnp.float32))
    sim_at_nearest = jnp.sum(jnp.where(iota == nearest_idx, sims, 0.0))
    sum_others = jnp.sum(jnp.where(mask, sims, 0.0)) - sim_at_nearest
    mean_others = sum_others / jnp.maximum(n_valid - 1.0, 1.0)
    contrast = jnp.where(n_valid > 1.0, sim_nearest - mean_others, 0.0)
    fire_mult = jnp.where(sim_nearest > FIRE_THRESH, FACIL_RATE, 1.0)
    facil_level = jnp.sum(
        jnp.where(iota == nearest_idx, facil_ref[...], 0.0)) * fire_mult
    selectivity = jax.nn.sigmoid(sharpness * contrast)
    gate = jnp.minimum(1.0 + k_gate * (facil_level - 1.0) * selectivity,
                       MAX_GATE)
    out_ref[0, 0] = gate


def _compute_gate(sums, buf, facil, mask_f, scal, n_rows):
    return pl.pallas_call(
        functools.partial(_gate_kernel, n_rows=n_rows),
        in_specs=[
            pl.BlockSpec(memory_space=pltpu.VMEM),
            pl.BlockSpec(memory_space=pltpu.VMEM),
            pl.BlockSpec(memory_space=pltpu.VMEM),
            pl.BlockSpec(memory_space=pltpu.VMEM),
            pl.BlockSpec(memory_space=pltpu.SMEM),
        ],
        out_specs=pl.BlockSpec(memory_space=pltpu.SMEM),
        out_shape=jax.ShapeDtypeStruct((1, 1), jnp.float32),
    )(sums, buf, facil, mask_f, scal)


_N_BUF = 512
_N_SC_WORKERS = 16
_ROWS_PER_W = _N_BUF // _N_SC_WORKERS
_L = 16


def _full(v, dtype=jnp.float32):
    return jnp.full((_L,), v, dtype=dtype)


def _vsum(vec):
    s = vec[0]
    for i in range(1, _L):
        s = s + vec[i]
    return s


def _vmax(vec):
    s = vec[0]
    for i in range(1, _L):
        s = jnp.maximum(s, vec[i])
    return s


def _sc_gate_body(q_hbm, buf_hbm, facil_hbm, mask_hbm, scal_hbm,
                  out_hbm, q_v, buf_v, maskf_v, scal_v, facil_v,
                  stage_v, all_v, out_v, sem, shared):
    w = jax.lax.axis_index("s")
    base_row = w * _ROWS_PER_W

    c1 = pltpu.async_copy(q_hbm, q_v, sem)
    c2 = pltpu.async_copy(
        buf_hbm.at[pl.ds(base_row * 1, _ROWS_PER_W), :], buf_v, sem)
    c3 = pltpu.async_copy(
        mask_hbm.at[pl.ds(base_row * 1, _ROWS_PER_W)], maskf_v, sem)
    c4 = pltpu.async_copy(scal_hbm, scal_v, sem)
    c1.wait()
    c2.wait()
    c3.wait()
    c4.wait()

    @pl.when(w == 0)
    def _copy_facil():
        pltpu.sync_copy(facil_hbm, facil_v)

    lanes = jax.lax.broadcasted_iota(jnp.int32, (_L,), 0)
    lmax = jnp.float32(-2.0)
    lidx = jnp.int32(0)
    lsum = jnp.float32(0.0)
    lcnt = jnp.float32(0.0)
    for k in range(_ROWS_PER_W // _L):
        simvec = jnp.zeros((_L,), jnp.float32)
        for j in range(_L):
            r = k * _L + j

            def dot_body(c, acc, _r=r):
                b = c * _L
                return acc + buf_v[_r, pl.ds(b, _L)] * q_v[0, pl.ds(b, _L)]

            acc = jax.lax.fori_loop(0, _D // _L, dot_body,
                                    jnp.zeros((_L,), jnp.float32), unroll=8)
            simvec = jnp.where(lanes == j, _full(0.0) + _vsum(acc),
                               simvec)

        sv = simvec
        mk = maskf_v[pl.ds(k * _L, _L)] > 0.5
        sm = jnp.where(mk, sv, _full(-1.0))
        cmax = _vmax(sm)
        lanei = jnp.int32(_L - 1)
        for i2 in range(_L - 1, -1, -1):
            lanei = jnp.where(sm[i2] == cmax, jnp.int32(i2), lanei)
        cidx = base_row + k * _L + lanei
        csum = _vsum(jnp.where(mk, sv, _full(0.0)))
        ccnt = _vsum(maskf_v[pl.ds(k * _L, _L)])
        take = cmax > lmax
        lmax = jnp.where(take, cmax, lmax)
        lidx = jnp.where(take, cidx, lidx)
        lsum = lsum + csum
        lcnt = lcnt + ccnt

    statv = jnp.where(lanes == 0, _full(0.0) + lmax,
            jnp.where(lanes == 1, _full(0.0) + lidx.astype(jnp.float32),
            jnp.where(lanes == 2, _full(0.0) + lsum,
            jnp.where(lanes == 3, _full(0.0) + lcnt, _full(0.0)))))
    stage_v[...] = statv
    pltpu.sync_copy(stage_v, shared.at[w])
    plsc.subcore_barrier()

    @pl.when(w == 0)
    def _finalize():
        pltpu.sync_copy(shared, all_v)
        gmax = jnp.float32(-2.0)
        gidx = jnp.int32(0)
        gsum = jnp.float32(0.0)
        gcnt = jnp.float32(0.0)
        for ww in range(_N_SC_WORKERS):
            row = all_v[ww, :]
            wmax = row[0]
            widx = row[1]
            gsum = gsum + row[2]
            gcnt = gcnt + row[3]
            take = wmax > gmax
            gmax = jnp.where(take, wmax, gmax)
            gidx = jnp.where(take, widx.astype(jnp.int32), gidx)

        fbase = jnp.bitwise_and(gidx, jnp.int32(~(_L - 1)))
        fwin = facil_v[pl.ds(fbase, _L)]
        flane = gidx - fbase
        fval = fwin[0]
        for i3 in range(1, _L):
            fval = jnp.where(flane == i3, fwin[i3], fval)
        fvec = _full(0.0) + fval
        gmaxv = _full(0.0) + gmax
        gsumv = _full(0.0) + gsum
        gcntv = _full(0.0) + gcnt
        meano = (gsumv - gmaxv) / jnp.maximum(gcntv - 1.0, _full(1.0))
        contrast = jnp.where(gcntv > 1.0, gmaxv - meano, _full(0.0))
        scal_vec = scal_v[...]
        kg = jnp.clip(jnp.exp(_full(0.0) + scal_vec[0]), 0.01, 5.0)
        sh = jnp.clip(jnp.exp(_full(0.0) + scal_vec[1]), 0.5, 20.0)
        fire = jnp.where(gmaxv > FIRE_THRESH, _full(FACIL_RATE), _full(1.0))
        flevel = fvec * fire
        sel = 1.0 / (1.0 + jnp.exp(-(sh * contrast)))
        gate = jnp.minimum(1.0 + kg * (flevel - 1.0) * sel, _full(MAX_GATE))
        out_v[...] = gate
        pltpu.sync_copy(out_v, out_hbm)


def _compute_gate_sc(q, buf, facil, mask_f, scal):
    mesh = plsc.VectorSubcoreMesh(
        core_axis_name="c", subcore_axis_name="s", num_cores=1)
    body = _sc_gate_body
    fn = pl.kernel(
        body,
        out_type=jax.ShapeDtypeStruct((_L,), jnp.float32),
        mesh=mesh,
        scratch_types=[
            pltpu.VMEM((1, _D), jnp.float32),
            pltpu.VMEM((_ROWS_PER_W, _D), jnp.float32),
            pltpu.VMEM((_ROWS_PER_W,), jnp.float32),
            pltpu.VMEM((_L,), jnp.float32),
            pltpu.VMEM((_N_BUF,), jnp.float32),
            pltpu.VMEM((_L,), jnp.float32),
            pltpu.VMEM((_N_SC_WORKERS, _L), jnp.float32),
            pltpu.VMEM((_L,), jnp.float32),
            pltpu.SemaphoreType.DMA,
            pltpu.VMEM_SHARED((_N_SC_WORKERS, _L), jnp.float32),
        ],
    )
    return fn(q, buf, facil, mask_f, scal)


def _scale_kernel(gate_ref, x_ref, out_ref):
    x = x_ref[...]
    a = (0.5 * gate_ref[0, 0]) * x
    out_ref[...] = a + a * jax.lax.erf(x * _INV_SQRT2)


def _scale(x2, gate):
    n_rows = x2.shape[0]
    grid = n_rows // _SCALE_BLOCK
    return pl.pallas_call(
        _scale_kernel,
        grid=(grid,),
        in_specs=[
            pl.BlockSpec(memory_space=pltpu.SMEM),
            pl.BlockSpec((_SCALE_BLOCK, _D), lambda i: (i, 0)),
        ],
        out_specs=pl.BlockSpec((_SCALE_BLOCK, _D), lambda i: (i, 0)),
        out_shape=jax.ShapeDtypeStruct((n_rows, _D), jnp.float32),
        compiler_params=pltpu.CompilerParams(
            dimension_semantics=("parallel",)),
    )(gate, x2)


def kernel(x, log_k_gate, log_sharpness, buf, facil, mask):
    orig_shape = x.shape
    x2 = x.reshape(-1, x.shape[-1])
    n_rows = x2.shape[0]

    q = _column_sums(x2)

    scal = jnp.zeros((_L,), jnp.float32)
    scal = scal.at[0].set(log_k_gate).at[1].set(log_sharpness)
    mask_f = mask.astype(jnp.float32)
    gate16 = _compute_gate_sc(q, buf, facil, mask_f, scal)
    gate = gate16[0:1].reshape(1, 1)

    out = _scale(x2, gate)
    return out.reshape(orig_shape)


# speedup vs baseline: 1.0528x; 1.0290x over previous
import math

import jax
import jax.numpy as jnp
from jax.experimental import pallas as pl
from jax.experimental.pallas import tpu as pltpu
from jax.experimental.pallas import tpu_sc as plsc

FACIL_RATE = 2.0
FIRE_THRESH = 0.85
MAX_GATE = 8.0

_INV_SQRT2 = 1.0 / math.sqrt(2.0)

_ROW_BLOCK = 4096
_SCALE_BLOCK = 2048
_D = 1024
_N_ROWS = 4 * 8192

_N_BUF = 512
_N_SC_WORKERS = 32
_ROWS_PER_W = _N_BUF // _N_SC_WORKERS
_L = 16


def _gelu(x):
    h = 0.5 * x
    return h + h * jax.lax.erf(x * _INV_SQRT2)


def _sum_kernel(x_ref, out_ref, acc_ref):
    i = pl.program_id(0)

    def body(j, carry):
        s1, s2 = carry
        xx = x_ref[pl.ds(j * 8, 8), :]
        e = jax.lax.erf(xx * _INV_SQRT2)
        return (s1 + xx, s2 + xx * e)

    zero = jnp.zeros((8, _D), jnp.float32)
    s1, s2 = jax.lax.fori_loop(
        0, _ROW_BLOCK // 8, body, (zero, zero), unroll=4)
    part = 0.5 * (s1 + s2)

    @pl.when(i == 0)
    def _init():
        acc_ref[...] = part

    @pl.when(i > 0)
    def _acc():
        acc_ref[...] += part

    @pl.when(i == pl.num_programs(0) - 1)
    def _emit():
        total = acc_ref[...]
        m = jnp.sum(total, axis=0, keepdims=True) * (1.0 / float(_N_ROWS))
        norm = jnp.sqrt(jnp.sum(m * m))
        out_ref[...] = m / jnp.maximum(norm, 1e-12)


def _compute_q(x2):
    n_rows = x2.shape[0]
    grid = n_rows // _ROW_BLOCK
    return pl.pallas_call(
        _sum_kernel,
        grid=(grid,),
        in_specs=[pl.BlockSpec((_ROW_BLOCK, _D), lambda i: (i, 0))],
        out_specs=pl.BlockSpec((1, _D), lambda i: (0, 0)),
        out_shape=jax.ShapeDtypeStruct((1, _D), jnp.float32),
        scratch_shapes=[pltpu.VMEM((8, _D), jnp.float32)],
        compiler_params=pltpu.CompilerParams(
            dimension_semantics=("arbitrary",)),
    )(x2)


def _full(v, dtype=jnp.float32):
    return jnp.full((_L,), v, dtype=dtype)


def _vsum(vec):
    s = vec[0]
    for i in range(1, _L):
        s = s + vec[i]
    return s


def _vmax(vec):
    s = vec[0]
    for i in range(1, _L):
        s = jnp.maximum(s, vec[i])
    return s


def _sc_stats_body(q_hbm, buf_hbm, mask_hbm, out_hbm,
                   q_v, buf_v, maskf_v, stage_v, sem):
    wid = jax.lax.axis_index("s") * 2 + jax.lax.axis_index("c")
    base_row = wid * _ROWS_PER_W

    c1 = pltpu.async_copy(q_hbm, q_v, sem)
    c2 = pltpu.async_copy(
        buf_hbm.at[pl.ds(base_row * 1, _ROWS_PER_W), :], buf_v, sem)
    c3 = pltpu.async_copy(
        mask_hbm.at[pl.ds(base_row * 1, _ROWS_PER_W)], maskf_v, sem)
    c1.wait()
    c2.wait()
    c3.wait()

    lanes = jax.lax.broadcasted_iota(jnp.int32, (_L,), 0)
    simvec = jnp.zeros((_L,), jnp.float32)
    for j in range(_ROWS_PER_W):

        def dot_body(c, acc, _j=j):
            b = c * _L
            return acc + buf_v[_j, pl.ds(b, _L)] * q_v[0, pl.ds(b, _L)]

        acc = jax.lax.fori_loop(0, _D // _L, dot_body,
                                jnp.zeros((_L,), jnp.float32), unroll=8)
        simvec = jnp.where(lanes == j, _full(0.0) + _vsum(acc), simvec)

    mk = maskf_v[...] > 0.5
    sm = jnp.where(mk, simvec, _full(-1.0))
    cmax = _vmax(sm)
    lanei = jnp.int32(_L - 1)
    for i2 in range(_L - 1, -1, -1):
        lanei = jnp.where(sm[i2] == cmax, jnp.int32(i2), lanei)
    cidx = base_row + lanei
    csum = _vsum(jnp.where(mk, simvec, _full(0.0)))
    ccnt = _vsum(maskf_v[...])

    statv = jnp.where(lanes == 0, _full(0.0) + cmax,
            jnp.where(lanes == 1, _full(0.0) + cidx.astype(jnp.float32),
            jnp.where(lanes == 2, _full(0.0) + csum,
            jnp.where(lanes == 3, _full(0.0) + ccnt, _full(0.0)))))
    stage_v[...] = statv
    pltpu.sync_copy(stage_v, out_hbm.at[wid])


def _compute_stats_sc(q, buf, mask_f):
    mesh = plsc.VectorSubcoreMesh(core_axis_name="c", subcore_axis_name="s")
    fn = pl.kernel(
        _sc_stats_body,
        out_type=jax.ShapeDtypeStruct((_N_SC_WORKERS, _L), jnp.float32),
        mesh=mesh,
        scratch_types=[
            pltpu.VMEM((1, _D), jnp.float32),
            pltpu.VMEM((_ROWS_PER_W, _D), jnp.float32),
            pltpu.VMEM((_ROWS_PER_W,), jnp.float32),
            pltpu.VMEM((_L,), jnp.float32),
            pltpu.SemaphoreType.DMA,
        ],
    )
    return fn(q, buf, mask_f)


def _scale_kernel(scal_ref, stats_ref, facil_ref, x_ref, out_ref, gate_ref):
    i = pl.program_id(0)

    @pl.when(i == 0)
    def _merge_gate():
        stats = stats_ref[...]
        wmax = stats[:, 0:1]
        widx = stats[:, 1:2]
        gmax = jnp.max(wmax)
        sel = wmax == gmax
        gidx_f = jnp.min(jnp.where(sel, widx, jnp.float32(2.0 ** 30)))
        gidx = gidx_f.astype(jnp.int32)
        gsum = jnp.sum(stats[:, 2:3])
        gcnt = jnp.sum(stats[:, 3:4])

        k_gate = jnp.clip(jnp.exp(scal_ref[0, 0]), 0.01, 5.0)
        sharpness = jnp.clip(jnp.exp(scal_ref[0, 1]), 0.5, 20.0)
        mean_others = (gsum - gmax) / jnp.maximum(gcnt - 1.0, 1.0)
        contrast = jnp.where(gcnt > 1.0, gmax - mean_others, 0.0)
        fire_mult = jnp.where(gmax > FIRE_THRESH, FACIL_RATE, 1.0)
        fiota = jax.lax.broadcasted_iota(jnp.int32, (1, _N_BUF), 1)
        facil_level = jnp.sum(
            jnp.where(fiota == gidx, facil_ref[...], 0.0)) * fire_mult
        selectivity = jax.nn.sigmoid(sharpness * contrast)
        gate = jnp.minimum(1.0 + k_gate * (facil_level - 1.0) * selectivity,
                           MAX_GATE)
        gate_ref[0, 0] = 0.5 * gate

    hg = gate_ref[0, 0]
    x = x_ref[...]
    a = hg * x
    out_ref[...] = a + a * jax.lax.erf(x * _INV_SQRT2)


def _scale(x2, scal, stats, facil2):
    n_rows = x2.shape[0]
    grid = n_rows // _SCALE_BLOCK
    return pl.pallas_call(
        _scale_kernel,
        grid=(grid,),
        in_specs=[
            pl.BlockSpec(memory_space=pltpu.SMEM),
            pl.BlockSpec((_N_SC_WORKERS, _L), lambda i: (0, 0)),
            pl.BlockSpec((1, _N_BUF), lambda i: (0, 0)),
            pl.BlockSpec((_SCALE_BLOCK, _D), lambda i: (i, 0)),
        ],
        out_specs=pl.BlockSpec((_SCALE_BLOCK, _D), lambda i: (i, 0)),
        out_shape=jax.ShapeDtypeStruct((n_rows, _D), jnp.float32),
        scratch_shapes=[pltpu.SMEM((1, 1), jnp.float32)],
        compiler_params=pltpu.CompilerParams(
            dimension_semantics=("arbitrary",)),
    )(scal, stats, facil2, x2)


def kernel(x, log_k_gate, log_sharpness, buf, facil, mask):
    orig_shape = x.shape
    x2 = x.reshape(-1, x.shape[-1])

    q = _compute_q(x2)
    mask_f = mask.astype(jnp.float32)
    stats = _compute_stats_sc(q, buf, mask_f)

    scal = jnp.stack([log_k_gate, log_sharpness]).reshape(1, 2)
    facil2 = facil.reshape(1, -1)
    out = _scale(x2, scal, stats, facil2)
    return out.reshape(orig_shape)


# speedup vs baseline: 1.0600x; 1.0068x over previous
import math

import jax
import jax.numpy as jnp
from jax.experimental import pallas as pl
from jax.experimental.pallas import tpu as pltpu
from jax.experimental.pallas import tpu_sc as plsc

FACIL_RATE = 2.0
FIRE_THRESH = 0.85
MAX_GATE = 8.0

_INV_SQRT2 = 1.0 / math.sqrt(2.0)

_ROW_BLOCK = 4096
_SCALE_BLOCK = 2048
_D = 1024
_N_ROWS = 4 * 8192

_N_BUF = 512
_N_SC_WORKERS = 32
_ROWS_PER_W = _N_BUF // _N_SC_WORKERS
_L = 16


def _gelu(x):
    h = 0.5 * x
    return h + h * jax.lax.erf(x * _INV_SQRT2)


def _sum_kernel(x_ref, out_ref, acc_ref):
    i = pl.program_id(0)

    def body(j, carry):
        s1, s2 = carry
        xx = x_ref[pl.ds(j * 8, 8), :]
        e = jax.lax.erf(xx * _INV_SQRT2)
        return (s1 + xx, s2 + xx * e)

    zero = jnp.zeros((8, _D), jnp.float32)
    s1, s2 = jax.lax.fori_loop(
        0, _ROW_BLOCK // 8, body, (zero, zero), unroll=4)
    part = 0.5 * (s1 + s2)

    @pl.when(i == 0)
    def _init():
        acc_ref[...] = part

    @pl.when(i > 0)
    def _acc():
        acc_ref[...] += part

    @pl.when(i == pl.num_programs(0) - 1)
    def _emit():
        total = acc_ref[...]
        m = jnp.sum(total, axis=0, keepdims=True) * (1.0 / float(_N_ROWS))
        norm = jnp.sqrt(jnp.sum(m * m))
        out_ref[...] = m / jnp.maximum(norm, 1e-12)


def _compute_q(x2):
    n_rows = x2.shape[0]
    grid = n_rows // _ROW_BLOCK
    return pl.pallas_call(
        _sum_kernel,
        grid=(grid,),
        in_specs=[pl.BlockSpec((_ROW_BLOCK, _D), lambda i: (i, 0))],
        out_specs=pl.BlockSpec((1, _D), lambda i: (0, 0)),
        out_shape=jax.ShapeDtypeStruct((1, _D), jnp.float32),
        scratch_shapes=[pltpu.VMEM((8, _D), jnp.float32)],
        compiler_params=pltpu.CompilerParams(
            dimension_semantics=("arbitrary",)),
    )(x2)


def _full(v, dtype=jnp.float32):
    return jnp.full((_L,), v, dtype=dtype)


def _vsum(vec):
    s = vec[0]
    for i in range(1, _L):
        s = s + vec[i]
    return s


def _vmax(vec):
    s = vec[0]
    for i in range(1, _L):
        s = jnp.maximum(s, vec[i])
    return s


def _sc_stats_body(q_hbm, buf_hbm, mask_hbm, out_hbm,
                   q_v, buf_v, maskf_v, stage_v, sem):
    wid = jax.lax.axis_index("s") * 2 + jax.lax.axis_index("c")
    base_row = wid * _ROWS_PER_W

    c1 = pltpu.async_copy(q_hbm, q_v, sem)
    c2 = pltpu.async_copy(
        buf_hbm.at[pl.ds(base_row * 1, _ROWS_PER_W), :], buf_v, sem)
    c3 = pltpu.async_copy(
        mask_hbm.at[pl.ds(base_row * 1, _ROWS_PER_W)], maskf_v, sem)
    c1.wait()
    c2.wait()
    c3.wait()

    lanes = jax.lax.broadcasted_iota(jnp.int32, (_L,), 0)

    def dot_body(c, accs):
        b = c * _L
        qc = q_v[0, pl.ds(b, _L)]
        return tuple(accs[j] + buf_v[j, pl.ds(b, _L)] * qc
                     for j in range(_ROWS_PER_W))

    zeros = tuple(jnp.zeros((_L,), jnp.float32) for _ in range(_ROWS_PER_W))
    accs = jax.lax.fori_loop(0, _D // _L, dot_body, zeros, unroll=2)
    simvec = jnp.zeros((_L,), jnp.float32)
    for j in range(_ROWS_PER_W):
        simvec = jnp.where(lanes == j, _full(0.0) + _vsum(accs[j]), simvec)

    mk = maskf_v[...] > 0.5
    sm = jnp.where(mk, simvec, _full(-1.0))
    cmax = _vmax(sm)
    lanei = jnp.int32(_L - 1)
    for i2 in range(_L - 1, -1, -1):
        lanei = jnp.where(sm[i2] == cmax, jnp.int32(i2), lanei)
    cidx = base_row + lanei
    csum = _vsum(jnp.where(mk, simvec, _full(0.0)))
    ccnt = _vsum(maskf_v[...])

    statv = jnp.where(lanes == 0, _full(0.0) + cmax,
            jnp.where(lanes == 1, _full(0.0) + cidx.astype(jnp.float32),
            jnp.where(lanes == 2, _full(0.0) + csum,
            jnp.where(lanes == 3, _full(0.0) + ccnt, _full(0.0)))))
    stage_v[...] = statv
    pltpu.sync_copy(stage_v, out_hbm.at[wid])


def _compute_stats_sc(q, buf, mask_f):
    mesh = plsc.VectorSubcoreMesh(core_axis_name="c", subcore_axis_name="s")
    fn = pl.kernel(
        _sc_stats_body,
        out_type=jax.ShapeDtypeStruct((_N_SC_WORKERS, _L), jnp.float32),
        mesh=mesh,
        scratch_types=[
            pltpu.VMEM((1, _D), jnp.float32),
            pltpu.VMEM((_ROWS_PER_W, _D), jnp.float32),
            pltpu.VMEM((_ROWS_PER_W,), jnp.float32),
            pltpu.VMEM((_L,), jnp.float32),
            pltpu.SemaphoreType.DMA,
        ],
    )
    return fn(q, buf, mask_f)


def _scale_kernel(scal_ref, stats_ref, facil_ref, x_ref, out_ref, gate_ref):
    i = pl.program_id(0)

    @pl.when(i == 0)
    def _merge_gate():
        stats = stats_ref[...]
        wmax = stats[:, 0:1]
        widx = stats[:, 1:2]
        gmax = jnp.max(wmax)
        sel = wmax == gmax
        gidx_f = jnp.min(jnp.where(sel, widx, jnp.float32(2.0 ** 30)))
        gidx = gidx_f.astype(jnp.int32)
        gsum = jnp.sum(stats[:, 2:3])
        gcnt = jnp.sum(stats[:, 3:4])

        k_gate = jnp.clip(jnp.exp(scal_ref[0, 0]), 0.01, 5.0)
        sharpness = jnp.clip(jnp.exp(scal_ref[0, 1]), 0.5, 20.0)
        mean_others = (gsum - gmax) / jnp.maximum(gcnt - 1.0, 1.0)
        contrast = jnp.where(gcnt > 1.0, gmax - mean_others, 0.0)
        fire_mult = jnp.where(gmax > FIRE_THRESH, FACIL_RATE, 1.0)
        fiota = jax.lax.broadcasted_iota(jnp.int32, (1, _N_BUF), 1)
        facil_level = jnp.sum(
            jnp.where(fiota == gidx, facil_ref[...], 0.0)) * fire_mult
        selectivity = jax.nn.sigmoid(sharpness * contrast)
        gate = jnp.minimum(1.0 + k_gate * (facil_level - 1.0) * selectivity,
                           MAX_GATE)
        gate_ref[0, 0] = 0.5 * gate

    hg = gate_ref[0, 0]
    x = x_ref[...]
    a = hg * x
    out_ref[...] = a + a * jax.lax.erf(x * _INV_SQRT2)


def _scale(x2, scal, stats, facil2):
    n_rows = x2.shape[0]
    grid = n_rows // _SCALE_BLOCK
    return pl.pallas_call(
        _scale_kernel,
        grid=(grid,),
        in_specs=[
            pl.BlockSpec(memory_space=pltpu.SMEM),
            pl.BlockSpec((_N_SC_WORKERS, _L), lambda i: (0, 0)),
            pl.BlockSpec((1, _N_BUF), lambda i: (0, 0)),
            pl.BlockSpec((_SCALE_BLOCK, _D), lambda i: (i, 0)),
        ],
        out_specs=pl.BlockSpec((_SCALE_BLOCK, _D), lambda i: (i, 0)),
        out_shape=jax.ShapeDtypeStruct((n_rows, _D), jnp.float32),
        scratch_shapes=[pltpu.SMEM((1, 1), jnp.float32)],
        compiler_params=pltpu.CompilerParams(
            dimension_semantics=("arbitrary",)),
    )(scal, stats, facil2, x2)


def kernel(x, log_k_gate, log_sharpness, buf, facil, mask):
    orig_shape = x.shape
    x2 = x.reshape(-1, x.shape[-1])

    q = _compute_q(x2)
    mask_f = mask.astype(jnp.float32)
    stats = _compute_stats_sc(q, buf, mask_f)

    scal = jnp.stack([log_k_gate, log_sharpness]).reshape(1, 2)
    facil2 = facil.reshape(1, -1)
    out = _scale(x2, scal, stats, facil2)
    return out.reshape(orig_shape)
